# Initial kernel scaffold; baseline (speedup 1.0000x reference)
import jax
import jax.numpy as jnp
from jax.experimental import pallas as pl


def kernel(x, edge_index, W1, b1, W2, b2, Ws1, bs1, alpha2, Ws3, bs3, alpha4, Wf1, bf1, Wf2, bf2, Wout, bout):
    raise NotImplementedError("write your pallas kernel here")


# speedup vs baseline: 10.3609x; 10.3609x over previous
import functools

import jax
import jax.numpy as jnp
from jax import lax
from jax.experimental import pallas as pl
from jax.experimental.pallas import tpu as pltpu
from jax.experimental.pallas import tpu_sc as plsc

N_NODES = 10000
F_IN = 128
D = 32
NC = 2
NS = 16
NW = NC * NS
CHUNK = 128
ROWS_PER_TILE = 632
N_PAD = NS * ROWS_PER_TILE
BLK = 1000


def _make_sc_segsum(chunks: int, do_deg: bool):
    mesh = plsc.VectorSubcoreMesh(
        core_axis_name="c", subcore_axis_name="s",
        num_cores=NC, num_subcores=NS)

    out_type = [jax.ShapeDtypeStruct((NC, N_PAD, D), jnp.float32)]
    scratch = [
        pltpu.VMEM((chunks, CHUNK), jnp.int32),
        pltpu.VMEM((chunks, CHUNK), jnp.int32),
        pltpu.VMEM((CHUNK, D), jnp.float32),
        pltpu.VMEM((ROWS_PER_TILE, D), jnp.float32),
        pltpu.VMEM_SHARED((N_PAD, D), jnp.float32),
        pltpu.SemaphoreType.DMA,
    ]
    if do_deg:
        out_type.append(jax.ShapeDtypeStruct((NC * N_PAD,), jnp.float32))
        scratch += [
            pltpu.VMEM((CHUNK,), jnp.float32),
            pltpu.VMEM((ROWS_PER_TILE,), jnp.float32),
            pltpu.VMEM_SHARED((N_PAD,), jnp.float32),
        ]

    def body(y_hbm, srcg, dstg, zeros2d, zeros1d, ones_h, *rest):
        if do_deg:
            (acc_out, deg_out, idx_s, idx_d, gbuf, vbuf, acc_sh, sem,
             ones_v, dbuf, deg_sh) = rest
        else:
            acc_out, idx_s, idx_d, gbuf, vbuf, acc_sh, sem = rest
        c = lax.axis_index("c")
        s = lax.axis_index("s")
        wid = s * NC + c
        row0 = s * ROWS_PER_TILE

        pltpu.sync_copy(zeros2d, vbuf)
        pltpu.sync_copy(vbuf, acc_sh.at[pl.ds(row0, ROWS_PER_TILE)])
        if do_deg:
            pltpu.sync_copy(zeros1d, dbuf)
            pltpu.sync_copy(dbuf, deg_sh.at[pl.ds(row0, ROWS_PER_TILE)])
            pltpu.sync_copy(ones_h, ones_v)
        pltpu.sync_copy(srcg.at[wid], idx_s)
        pltpu.sync_copy(dstg.at[wid], idx_d)
        plsc.subcore_barrier()

        def step(j, carry):
            pltpu.async_copy(y_hbm.at[idx_s.at[j]], gbuf, sem).wait()
            pltpu.sync_copy(gbuf, acc_sh.at[idx_d.at[j]], add=True)
            if do_deg:
                pltpu.sync_copy(ones_v, deg_sh.at[idx_d.at[j]], add=True)
            return carry

        lax.fori_loop(0, chunks, step, 0)
        plsc.subcore_barrier()

        pltpu.sync_copy(acc_sh.at[pl.ds(row0, ROWS_PER_TILE)], vbuf)
        pltpu.sync_copy(vbuf, acc_out.at[c, pl.ds(row0, ROWS_PER_TILE)])
        if do_deg:
            pltpu.sync_copy(deg_sh.at[pl.ds(row0, ROWS_PER_TILE)], dbuf)
            pltpu.sync_copy(
                dbuf, deg_out.at[pl.ds(c * N_PAD + row0, ROWS_PER_TILE)])

    return pl.kernel(body, out_type=tuple(out_type), mesh=mesh,
                     scratch_types=scratch,
                     compiler_params=pltpu.CompilerParams(
                         use_tc_tiling_on_sc=False))


def _proj_body(x_ref, w_ref, o_ref):
    o_ref[...] = jnp.dot(x_ref[...], w_ref[...],
                         preferred_element_type=jnp.float32)


def _mid_body(p0, p1, d0, d1, b1, w2, o_ref):
    deg = jnp.clip(d0[...] + d1[...], 1.0, None)
    h = jnp.maximum((p0[...] + p1[...]) / deg + b1[...], 0.0)
    o_ref[...] = jnp.dot(h, w2[...], preferred_element_type=jnp.float32)


def _tail_body(p0, p1, d0, d1, b2, ws1, bs1, a2, ws3, bs3, a4,
               wf1, bf1, wf2, bf2, wout, bout, o_ref):
    deg = jnp.clip(d0[...] + d1[...], 1.0, None)
    s = jnp.maximum((p0[...] + p1[...]) / deg + b2[...], 0.0)
    s = a2[0, 0] * jnp.maximum(
        jnp.dot(s, ws1[...], preferred_element_type=jnp.float32) + bs1[...], 0.0)
    s = a4[0, 0] * jnp.maximum(
        jnp.dot(s, ws3[...], preferred_element_type=jnp.float32) + bs3[...], 0.0)
    s = jnp.maximum(
        jnp.dot(s, wf1[...], preferred_element_type=jnp.float32) + bf1[...], 0.0)
    s = jnp.maximum(
        jnp.dot(s, wf2[...], preferred_element_type=jnp.float32) + bf2[...], 0.0)
    o_ref[...] = jnp.dot(s, wout[...],
                         preferred_element_type=jnp.float32) + bout[...]


def kernel(x, edge_index, W1, b1, W2, b2, Ws1, bs1, alpha2, Ws3, bs3,
           alpha4, Wf1, bf1, Wf2, bf2, Wout, bout):
    n = x.shape[0]
    e = edge_index.shape[1]
    grid = n // BLK

    chunks = -(-e // (NW * CHUNK))
    e_pad = NW * chunks * CHUNK
    src = jnp.concatenate(
        [edge_index[0], jnp.zeros((e_pad - e,), jnp.int32)]).reshape(
            NW, chunks, CHUNK)
    dst = jnp.concatenate(
        [edge_index[1], jnp.full((e_pad - e,), n, jnp.int32)]).reshape(
            NW, chunks, CHUNK)
    zeros2d = jnp.zeros((ROWS_PER_TILE, D), jnp.float32)
    zeros1d = jnp.zeros((ROWS_PER_TILE,), jnp.float32)
    ones_h = jnp.ones((CHUNK,), jnp.float32)

    y1 = pl.pallas_call(
        _proj_body,
        grid=(grid,),
        in_specs=[pl.BlockSpec((BLK, F_IN), lambda i: (i, 0)),
                  pl.BlockSpec((F_IN, D), lambda i: (0, 0))],
        out_specs=pl.BlockSpec((BLK, D), lambda i: (i, 0)),
        out_shape=jax.ShapeDtypeStruct((n, D), jnp.float32))(x, W1)

    seg1 = _make_sc_segsum(chunks, True)
    acc1, degp = seg1(y1, src, dst, zeros2d, zeros1d, ones_h)
    degp = degp.reshape(NC, N_PAD)
    p0, p1 = acc1[0, :n], acc1[1, :n]
    d0, d1 = degp[0, :n, None], degp[1, :n, None]

    row_spec = pl.BlockSpec((BLK, D), lambda i: (i, 0))
    deg_spec = pl.BlockSpec((BLK, 1), lambda i: (i, 0))
    w_spec = pl.BlockSpec((D, D), lambda i: (0, 0))
    b_spec = pl.BlockSpec((1, D), lambda i: (0, 0))
    a_spec = pl.BlockSpec((1, 1), lambda i: (0, 0))

    y2 = pl.pallas_call(
        _mid_body,
        grid=(grid,),
        in_specs=[row_spec, row_spec, deg_spec, deg_spec, b_spec, w_spec],
        out_specs=row_spec,
        out_shape=jax.ShapeDtypeStruct((n, D), jnp.float32))(
            p0, p1, d0, d1, b1[None, :], W2)

    seg2 = _make_sc_segsum(chunks, False)
    (acc2,) = seg2(y2, src, dst, zeros2d, zeros1d, ones_h)
    q0, q1 = acc2[0, :n], acc2[1, :n]

    c = Wout.shape[1]
    out = pl.pallas_call(
        _tail_body,
        grid=(grid,),
        in_specs=[row_spec, row_spec, deg_spec, deg_spec, b_spec,
                  w_spec, b_spec, a_spec, w_spec, b_spec, a_spec,
                  w_spec, b_spec, w_spec, b_spec,
                  pl.BlockSpec((D, c), lambda i: (0, 0)),
                  pl.BlockSpec((1, c), lambda i: (0, 0))],
        out_specs=pl.BlockSpec((BLK, c), lambda i: (i, 0)),
        out_shape=jax.ShapeDtypeStruct((n, c), jnp.float32))(
            q0, q1, d0, d1, b2[None, :], Ws1, bs1[None, :],
            alpha2[:, None], Ws3, bs3[None, :], alpha4[:, None],
            Wf1, bf1[None, :], Wf2, bf2[None, :], Wout, bout[None, :])
    return out


# speedup vs baseline: 11.1332x; 1.0745x over previous
import functools

import jax
import jax.numpy as jnp
from jax import lax
from jax.experimental import pallas as pl
from jax.experimental.pallas import tpu as pltpu
from jax.experimental.pallas import tpu_sc as plsc

N_NODES = 10000
F_IN = 128
D = 32
NC = 2
NS = 16
NW = NC * NS
CHUNK = 128
KG = 4
ROWS_PER_TILE = 632
N_PAD = NS * ROWS_PER_TILE
BLK = 1000


def _make_sc_segsum(chunks: int, do_deg: bool):
    mesh = plsc.VectorSubcoreMesh(
        core_axis_name="c", subcore_axis_name="s",
        num_cores=NC, num_subcores=NS)

    ngroups = chunks // KG
    out_type = [jax.ShapeDtypeStruct((NC, N_PAD, D), jnp.float32)]
    scratch = [
        pltpu.VMEM((chunks, CHUNK), jnp.int32),
        pltpu.VMEM((chunks, CHUNK), jnp.int32),
        pltpu.VMEM((2, KG, CHUNK, D), jnp.float32),
        pltpu.VMEM((ROWS_PER_TILE, D), jnp.float32),
        pltpu.VMEM_SHARED((N_PAD, D), jnp.float32),
        pltpu.SemaphoreType.DMA,
        pltpu.SemaphoreType.DMA,
    ]
    if do_deg:
        out_type.append(jax.ShapeDtypeStruct((NC * N_PAD,), jnp.float32))
        scratch += [
            pltpu.VMEM((CHUNK,), jnp.float32),
            pltpu.VMEM((ROWS_PER_TILE,), jnp.float32),
            pltpu.VMEM_SHARED((N_PAD,), jnp.float32),
            pltpu.SemaphoreType.DMA,
        ]

    def body(y_hbm, srcg, dstg, zeros2d, zeros1d, ones_h, *rest):
        if do_deg:
            (acc_out, deg_out, idx_s, idx_d, gbufs, vbuf, acc_sh, gsem,
             ssem, ones_v, dbuf, deg_sh, dsem) = rest
        else:
            acc_out, idx_s, idx_d, gbufs, vbuf, acc_sh, gsem, ssem = rest
        c = lax.axis_index("c")
        s = lax.axis_index("s")
        wid = s * NC + c
        row0 = s * ROWS_PER_TILE

        pltpu.sync_copy(srcg.at[wid], idx_s)
        pltpu.sync_copy(dstg.at[wid], idx_d)

        def fire_gathers(g, p):
            for b in range(KG):
                pltpu.async_copy(y_hbm.at[idx_s.at[g * KG + b]],
                                 gbufs.at[p, b], gsem)

        fire_gathers(0, 0)

        pltpu.sync_copy(zeros2d, vbuf)
        pltpu.sync_copy(vbuf, acc_sh.at[pl.ds(row0, ROWS_PER_TILE)])
        if do_deg:
            pltpu.sync_copy(zeros1d, dbuf)
            pltpu.sync_copy(dbuf, deg_sh.at[pl.ds(row0, ROWS_PER_TILE)])
            pltpu.sync_copy(ones_h, ones_v)
        plsc.subcore_barrier()

        def do_group(g, p):
            for b in range(KG):
                pltpu.make_async_copy(y_hbm.at[idx_s.at[g * KG + b]],
                                      gbufs.at[p, b], gsem).wait()
            for b in range(KG):
                pltpu.async_copy(gbufs.at[p, b],
                                 acc_sh.at[idx_d.at[g * KG + b]],
                                 ssem, add=True)
                if do_deg:
                    pltpu.async_copy(ones_v,
                                     deg_sh.at[idx_d.at[g * KG + b]],
                                     dsem, add=True)
            for b in range(KG):
                pltpu.make_async_copy(gbufs.at[p, b],
                                      acc_sh.at[idx_d.at[g * KG + b]],
                                      ssem).wait()
                if do_deg:
                    pltpu.make_async_copy(ones_v,
                                          deg_sh.at[idx_d.at[g * KG + b]],
                                          dsem).wait()

        def step(g, carry):
            fire_gathers(g + 1, lax.rem(g + 1, 2))
            do_group(g, lax.rem(g, 2))
            return carry

        lax.fori_loop(0, ngroups - 1, step, 0)
        do_group(ngroups - 1, (ngroups - 1) % 2)
        plsc.subcore_barrier()

        pltpu.sync_copy(acc_sh.at[pl.ds(row0, ROWS_PER_TILE)], vbuf)
        pltpu.sync_copy(vbuf, acc_out.at[c, pl.ds(row0, ROWS_PER_TILE)])
        if do_deg:
            pltpu.sync_copy(deg_sh.at[pl.ds(row0, ROWS_PER_TILE)], dbuf)
            pltpu.sync_copy(
                dbuf, deg_out.at[pl.ds(c * N_PAD + row0, ROWS_PER_TILE)])

    return pl.kernel(body, out_type=tuple(out_type), mesh=mesh,
                     scratch_types=scratch,
                     compiler_params=pltpu.CompilerParams(
                         use_tc_tiling_on_sc=False))


def _proj_body(x_ref, w_ref, o_ref):
    o_ref[...] = jnp.dot(x_ref[...], w_ref[...],
                         preferred_element_type=jnp.float32)


def _mid_body(p0, p1, d0, d1, b1, w2, o_ref):
    deg = jnp.clip(d0[...] + d1[...], 1.0, None)
    h = jnp.maximum((p0[...] + p1[...]) / deg + b1[...], 0.0)
    o_ref[...] = jnp.dot(h, w2[...], preferred_element_type=jnp.float32)


def _tail_body(p0, p1, d0, d1, b2, ws1, bs1, a2, ws3, bs3, a4,
               wf1, bf1, wf2, bf2, wout, bout, o_ref):
    deg = jnp.clip(d0[...] + d1[...], 1.0, None)
    s = jnp.maximum((p0[...] + p1[...]) / deg + b2[...], 0.0)
    s = a2[0, 0] * jnp.maximum(
        jnp.dot(s, ws1[...], preferred_element_type=jnp.float32) + bs1[...], 0.0)
    s = a4[0, 0] * jnp.maximum(
        jnp.dot(s, ws3[...], preferred_element_type=jnp.float32) + bs3[...], 0.0)
    s = jnp.maximum(
        jnp.dot(s, wf1[...], preferred_element_type=jnp.float32) + bf1[...], 0.0)
    s = jnp.maximum(
        jnp.dot(s, wf2[...], preferred_element_type=jnp.float32) + bf2[...], 0.0)
    o_ref[...] = jnp.dot(s, wout[...],
                         preferred_element_type=jnp.float32) + bout[...]


def kernel(x, edge_index, W1, b1, W2, b2, Ws1, bs1, alpha2, Ws3, bs3,
           alpha4, Wf1, bf1, Wf2, bf2, Wout, bout):
    n = x.shape[0]
    e = edge_index.shape[1]
    grid = n // BLK

    chunks = -(-e // (NW * CHUNK))
    chunks = -(-chunks // KG) * KG
    e_pad = NW * chunks * CHUNK
    src = jnp.concatenate(
        [edge_index[0], jnp.zeros((e_pad - e,), jnp.int32)]).reshape(
            NW, chunks, CHUNK)
    dst = jnp.concatenate(
        [edge_index[1], jnp.full((e_pad - e,), n, jnp.int32)]).reshape(
            NW, chunks, CHUNK)
    zeros2d = jnp.zeros((ROWS_PER_TILE, D), jnp.float32)
    zeros1d = jnp.zeros((ROWS_PER_TILE,), jnp.float32)
    ones_h = jnp.ones((CHUNK,), jnp.float32)

    y1 = pl.pallas_call(
        _proj_body,
        grid=(grid,),
        in_specs=[pl.BlockSpec((BLK, F_IN), lambda i: (i, 0)),
                  pl.BlockSpec((F_IN, D), lambda i: (0, 0))],
        out_specs=pl.BlockSpec((BLK, D), lambda i: (i, 0)),
        out_shape=jax.ShapeDtypeStruct((n, D), jnp.float32))(x, W1)

    seg1 = _make_sc_segsum(chunks, True)
    acc1, degp = seg1(y1, src, dst, zeros2d, zeros1d, ones_h)
    degp = degp.reshape(NC, N_PAD)
    p0, p1 = acc1[0, :n], acc1[1, :n]
    d0, d1 = degp[0, :n, None], degp[1, :n, None]

    row_spec = pl.BlockSpec((BLK, D), lambda i: (i, 0))
    deg_spec = pl.BlockSpec((BLK, 1), lambda i: (i, 0))
    w_spec = pl.BlockSpec((D, D), lambda i: (0, 0))
    b_spec = pl.BlockSpec((1, D), lambda i: (0, 0))
    a_spec = pl.BlockSpec((1, 1), lambda i: (0, 0))

    y2 = pl.pallas_call(
        _mid_body,
        grid=(grid,),
        in_specs=[row_spec, row_spec, deg_spec, deg_spec, b_spec, w_spec],
        out_specs=row_spec,
        out_shape=jax.ShapeDtypeStruct((n, D), jnp.float32))(
            p0, p1, d0, d1, b1[None, :], W2)

    seg2 = _make_sc_segsum(chunks, False)
    (acc2,) = seg2(y2, src, dst, zeros2d, zeros1d, ones_h)
    q0, q1 = acc2[0, :n], acc2[1, :n]

    c = Wout.shape[1]
    out = pl.pallas_call(
        _tail_body,
        grid=(grid,),
        in_specs=[row_spec, row_spec, deg_spec, deg_spec, b_spec,
                  w_spec, b_spec, a_spec, w_spec, b_spec, a_spec,
                  w_spec, b_spec, w_spec, b_spec,
                  pl.BlockSpec((D, c), lambda i: (0, 0)),
                  pl.BlockSpec((1, c), lambda i: (0, 0))],
        out_specs=pl.BlockSpec((BLK, c), lambda i: (i, 0)),
        out_shape=jax.ShapeDtypeStruct((n, c), jnp.float32))(
            q0, q1, d0, d1, b2[None, :], Ws1, bs1[None, :],
            alpha2[:, None], Ws3, bs3[None, :], alpha4[:, None],
            Wf1, bf1[None, :], Wf2, bf2[None, :], Wout, bout[None, :])
    return out


# speedup vs baseline: 17.0661x; 1.5329x over previous
import functools

import jax
import jax.numpy as jnp
from jax import lax
from jax.experimental import pallas as pl
from jax.experimental.pallas import tpu as pltpu
from jax.experimental.pallas import tpu_sc as plsc

N_NODES = 10000
F_IN = 128
D = 32
NC = 2
NS = 16
NW = NC * NS
CHUNK = 128
KG = 4
ROWS_PER_TILE = 632
SROWS = N_NODES // NS
N_PAD = NS * ROWS_PER_TILE
BLK = 1000


def _make_sc_segsum(chunks: int, do_deg: bool):
    mesh = plsc.VectorSubcoreMesh(
        core_axis_name="c", subcore_axis_name="s",
        num_cores=NC, num_subcores=NS)

    ngroups = chunks // KG
    out_type = [jax.ShapeDtypeStruct((NC, N_PAD, D), jnp.float32)]
    scratch = [
        pltpu.VMEM((chunks, CHUNK), jnp.int32),
        pltpu.VMEM((chunks, CHUNK), jnp.int32),
        pltpu.VMEM((2, KG, CHUNK, D), jnp.float32),
        pltpu.VMEM((ROWS_PER_TILE, D), jnp.float32),
        pltpu.VMEM_SHARED((N_PAD, D), jnp.float32),
        pltpu.VMEM_SHARED((N_NODES, D), jnp.float32),
        pltpu.SemaphoreType.DMA,
        pltpu.SemaphoreType.DMA,
    ]
    if do_deg:
        out_type.append(jax.ShapeDtypeStruct((NC * N_PAD,), jnp.float32))
        scratch += [
            pltpu.VMEM((CHUNK,), jnp.float32),
            pltpu.VMEM((ROWS_PER_TILE,), jnp.float32),
            pltpu.VMEM_SHARED((N_PAD,), jnp.float32),
            pltpu.SemaphoreType.DMA,
        ]

    def body(y_hbm, srcg, dstg, zeros2d, zeros1d, ones_h, *rest):
        if do_deg:
            (acc_out, deg_out, idx_s, idx_d, gbufs, vbuf, acc_sh, y_sh,
             gsem, ssem, ones_v, dbuf, deg_sh, dsem) = rest
        else:
            (acc_out, idx_s, idx_d, gbufs, vbuf, acc_sh, y_sh,
             gsem, ssem) = rest
        c = lax.axis_index("c")
        s = lax.axis_index("s")
        wid = s * NC + c
        row0 = s * ROWS_PER_TILE

        pltpu.sync_copy(srcg.at[wid], idx_s)
        pltpu.sync_copy(dstg.at[wid], idx_d)

        def fire_gathers(g, p):
            for b in range(KG):
                pltpu.async_copy(y_sh.at[idx_s.at[g * KG + b]],
                                 gbufs.at[p, b], gsem)

        srow = s * SROWS
        pltpu.sync_copy(y_hbm.at[pl.ds(srow, SROWS)],
                        vbuf.at[pl.ds(0, SROWS)])
        pltpu.sync_copy(vbuf.at[pl.ds(0, SROWS)],
                        y_sh.at[pl.ds(srow, SROWS)])
        pltpu.sync_copy(zeros2d, vbuf)
        pltpu.sync_copy(vbuf, acc_sh.at[pl.ds(row0, ROWS_PER_TILE)])
        if do_deg:
            pltpu.sync_copy(zeros1d, dbuf)
            pltpu.sync_copy(dbuf, deg_sh.at[pl.ds(row0, ROWS_PER_TILE)])
            pltpu.sync_copy(ones_h, ones_v)
        plsc.subcore_barrier()
        fire_gathers(0, 0)

        def do_group(g, p):
            for b in range(KG):
                pltpu.make_async_copy(y_sh.at[idx_s.at[g * KG + b]],
                                      gbufs.at[p, b], gsem).wait()
            for b in range(KG):
                pltpu.async_copy(gbufs.at[p, b],
                                 acc_sh.at[idx_d.at[g * KG + b]],
                                 ssem, add=True)
                if do_deg:
                    pltpu.async_copy(ones_v,
                                     deg_sh.at[idx_d.at[g * KG + b]],
                                     dsem, add=True)
            for b in range(KG):
                pltpu.make_async_copy(gbufs.at[p, b],
                                      acc_sh.at[idx_d.at[g * KG + b]],
                                      ssem).wait()
                if do_deg:
                    pltpu.make_async_copy(ones_v,
                                          deg_sh.at[idx_d.at[g * KG + b]],
                                          dsem).wait()

        def step(g, carry):
            fire_gathers(g + 1, lax.rem(g + 1, 2))
            do_group(g, lax.rem(g, 2))
            return carry

        lax.fori_loop(0, ngroups - 1, step, 0)
        do_group(ngroups - 1, (ngroups - 1) % 2)
        plsc.subcore_barrier()

        pltpu.sync_copy(acc_sh.at[pl.ds(row0, ROWS_PER_TILE)], vbuf)
        pltpu.sync_copy(vbuf, acc_out.at[c, pl.ds(row0, ROWS_PER_TILE)])
        if do_deg:
            pltpu.sync_copy(deg_sh.at[pl.ds(row0, ROWS_PER_TILE)], dbuf)
            pltpu.sync_copy(
                dbuf, deg_out.at[pl.ds(c * N_PAD + row0, ROWS_PER_TILE)])

    return pl.kernel(body, out_type=tuple(out_type), mesh=mesh,
                     scratch_types=scratch,
                     compiler_params=pltpu.CompilerParams(
                         use_tc_tiling_on_sc=False))


def _proj_body(x_ref, w_ref, o_ref):
    o_ref[...] = jnp.dot(x_ref[...], w_ref[...],
                         preferred_element_type=jnp.float32)


def _mid_body(p0, p1, d0, d1, b1, w2, o_ref):
    deg = jnp.clip(d0[...] + d1[...], 1.0, None)
    h = jnp.maximum((p0[...] + p1[...]) / deg + b1[...], 0.0)
    o_ref[...] = jnp.dot(h, w2[...], preferred_element_type=jnp.float32)


def _tail_body(p0, p1, d0, d1, b2, ws1, bs1, a2, ws3, bs3, a4,
               wf1, bf1, wf2, bf2, wout, bout, o_ref):
    deg = jnp.clip(d0[...] + d1[...], 1.0, None)
    s = jnp.maximum((p0[...] + p1[...]) / deg + b2[...], 0.0)
    s = a2[0, 0] * jnp.maximum(
        jnp.dot(s, ws1[...], preferred_element_type=jnp.float32) + bs1[...], 0.0)
    s = a4[0, 0] * jnp.maximum(
        jnp.dot(s, ws3[...], preferred_element_type=jnp.float32) + bs3[...], 0.0)
    s = jnp.maximum(
        jnp.dot(s, wf1[...], preferred_element_type=jnp.float32) + bf1[...], 0.0)
    s = jnp.maximum(
        jnp.dot(s, wf2[...], preferred_element_type=jnp.float32) + bf2[...], 0.0)
    o_ref[...] = jnp.dot(s, wout[...],
                         preferred_element_type=jnp.float32) + bout[...]


def kernel(x, edge_index, W1, b1, W2, b2, Ws1, bs1, alpha2, Ws3, bs3,
           alpha4, Wf1, bf1, Wf2, bf2, Wout, bout):
    n = x.shape[0]
    e = edge_index.shape[1]
    grid = n // BLK

    chunks = -(-e // (NW * CHUNK))
    chunks = -(-chunks // KG) * KG
    e_pad = NW * chunks * CHUNK
    src = jnp.concatenate(
        [edge_index[0], jnp.zeros((e_pad - e,), jnp.int32)]).reshape(
            NW, chunks, CHUNK)
    dst = jnp.concatenate(
        [edge_index[1], jnp.full((e_pad - e,), n, jnp.int32)]).reshape(
            NW, chunks, CHUNK)
    zeros2d = jnp.zeros((ROWS_PER_TILE, D), jnp.float32)
    zeros1d = jnp.zeros((ROWS_PER_TILE,), jnp.float32)
    ones_h = jnp.ones((CHUNK,), jnp.float32)

    y1 = pl.pallas_call(
        _proj_body,
        grid=(grid,),
        in_specs=[pl.BlockSpec((BLK, F_IN), lambda i: (i, 0)),
                  pl.BlockSpec((F_IN, D), lambda i: (0, 0))],
        out_specs=pl.BlockSpec((BLK, D), lambda i: (i, 0)),
        out_shape=jax.ShapeDtypeStruct((n, D), jnp.float32))(x, W1)

    seg1 = _make_sc_segsum(chunks, True)
    acc1, degp = seg1(y1, src, dst, zeros2d, zeros1d, ones_h)
    degp = degp.reshape(NC, N_PAD)
    p0, p1 = acc1[0, :n], acc1[1, :n]
    d0, d1 = degp[0, :n, None], degp[1, :n, None]

    row_spec = pl.BlockSpec((BLK, D), lambda i: (i, 0))
    deg_spec = pl.BlockSpec((BLK, 1), lambda i: (i, 0))
    w_spec = pl.BlockSpec((D, D), lambda i: (0, 0))
    b_spec = pl.BlockSpec((1, D), lambda i: (0, 0))
    a_spec = pl.BlockSpec((1, 1), lambda i: (0, 0))

    y2 = pl.pallas_call(
        _mid_body,
        grid=(grid,),
        in_specs=[row_spec, row_spec, deg_spec, deg_spec, b_spec, w_spec],
        out_specs=row_spec,
        out_shape=jax.ShapeDtypeStruct((n, D), jnp.float32))(
            p0, p1, d0, d1, b1[None, :], W2)

    seg2 = _make_sc_segsum(chunks, False)
    (acc2,) = seg2(y2, src, dst, zeros2d, zeros1d, ones_h)
    q0, q1 = acc2[0, :n], acc2[1, :n]

    c = Wout.shape[1]
    out = pl.pallas_call(
        _tail_body,
        grid=(grid,),
        in_specs=[row_spec, row_spec, deg_spec, deg_spec, b_spec,
                  w_spec, b_spec, a_spec, w_spec, b_spec, a_spec,
                  w_spec, b_spec, w_spec, b_spec,
                  pl.BlockSpec((D, c), lambda i: (0, 0)),
                  pl.BlockSpec((1, c), lambda i: (0, 0))],
        out_specs=pl.BlockSpec((BLK, c), lambda i: (i, 0)),
        out_shape=jax.ShapeDtypeStruct((n, c), jnp.float32))(
            q0, q1, d0, d1, b2[None, :], Ws1, bs1[None, :],
            alpha2[:, None], Ws3, bs3[None, :], alpha4[:, None],
            Wf1, bf1[None, :], Wf2, bf2[None, :], Wout, bout[None, :])
    return out
